# baseline (speedup 1.0000x reference)
import functools

import jax
import jax.numpy as jnp
from jax.experimental import pallas as pl
from jax.experimental.pallas import tpu as pltpu

D_MODEL = 768
D_FF = 768
N_EXP = 64
N_TOK = 128


def _moe_kernel(hex_ref, x_ref, win_ref, wout_ref, bin_ref, bout_ref,
                out_ref, g_ref, h_ref):
    e = pl.program_id(0)
    p = pl.program_id(1)

    @pl.when((e == 0) & (p == 0))
    def _routing():
        hw = hex_ref[...]
        cols = jax.lax.broadcasted_iota(jnp.int32, hw.shape, 1)
        m1 = jnp.max(hw, axis=1, keepdims=True)
        a1 = jnp.min(jnp.where(hw == m1, cols, N_EXP), axis=1, keepdims=True)
        sel1 = cols == a1
        masked = jnp.where(sel1, -jnp.inf, hw)
        m2 = jnp.max(masked, axis=1, keepdims=True)
        a2 = jnp.min(jnp.where(masked == m2, cols, N_EXP), axis=1,
                     keepdims=True)
        s = m1 + m2 + 1e-8
        g_ref[...] = jnp.where(sel1, m1 / s, 0.0) + jnp.where(
            cols == a2, m2 / s, 0.0)
        out_ref[...] = jnp.zeros_like(out_ref)

    @pl.when(p == 0)
    def _in_proj():
        x = x_ref[...].astype(jnp.bfloat16)
        h = jax.lax.dot_general(x, win_ref[0].astype(jnp.bfloat16),
                                (((1,), (1,)), ((), ())),
                                preferred_element_type=jnp.float32)
        h = h + bin_ref[0]
        h_ref[...] = h * jax.lax.logistic(h)

    @pl.when(p == 1)
    def _out_proj():
        o = jax.lax.dot_general(h_ref[...].astype(jnp.bfloat16),
                                wout_ref[0].astype(jnp.bfloat16),
                                (((1,), (1,)), ((), ())),
                                preferred_element_type=jnp.float32)
        o = o + bout_ref[0]
        cols = jax.lax.broadcasted_iota(jnp.int32, (N_TOK, N_EXP), 1)
        g_col = jnp.sum(jnp.where(cols == e, g_ref[...], 0.0), axis=1,
                        keepdims=True)
        out_ref[...] += g_col * o


@functools.partial(jax.jit, static_argnames=("interpret",))
def kernel(x, hex_weights, W_in, W_out, bias_in, bias_out, interpret=False):
    Bb, Tt, D = x.shape
    x_flat = x.reshape(Bb * Tt, D)
    hex_flat = hex_weights.reshape(Bb * Tt, N_EXP)

    out = pl.pallas_call(
        _moe_kernel,
        grid=(N_EXP, 2),
        in_specs=[
            pl.BlockSpec((N_TOK, N_EXP), lambda e, p: (0, 0)),
            pl.BlockSpec((N_TOK, D_MODEL), lambda e, p: (0, 0)),
            pl.BlockSpec((1, D_FF, D_MODEL), lambda e, p: (e, 0, 0)),
            pl.BlockSpec((1, D_MODEL, D_FF),
                         lambda e, p: (jnp.maximum(2 * e + p - 1, 0) // 2,
                                       0, 0)),
            pl.BlockSpec((1, 1, D_FF), lambda e, p: (e, 0, 0)),
            pl.BlockSpec((1, 1, D_MODEL), lambda e, p: (e, 0, 0)),
        ],
        out_specs=pl.BlockSpec((N_TOK, D_MODEL), lambda e, p: (0, 0)),
        out_shape=jax.ShapeDtypeStruct((N_TOK, D_MODEL), jnp.float32),
        scratch_shapes=[pltpu.VMEM((N_TOK, N_EXP), jnp.float32),
                        pltpu.VMEM((N_TOK, D_FF), jnp.float32)],
        compiler_params=pltpu.CompilerParams(
            dimension_semantics=("arbitrary", "arbitrary")),
        interpret=interpret,
    )(hex_flat, x_flat, W_in, W_out,
      bias_in.reshape(N_EXP, 1, D_FF), bias_out.reshape(N_EXP, 1, D_MODEL))
    return out.reshape(Bb, Tt, D)


# speedup vs baseline: 1.6345x; 1.6345x over previous
import functools

import jax
import jax.numpy as jnp
from jax.experimental import pallas as pl
from jax.experimental.pallas import tpu as pltpu

D_MODEL = 768
D_FF = 768
N_EXP = 64
N_TOK = 128
E_BLK = 2


def _moe_kernel(hex_ref, x_ref, win_ref, wout_ref, bin_ref, bout_ref,
                out_ref, g_ref):
    i = pl.program_id(0)

    @pl.when(i == 0)
    def _routing():
        hw = hex_ref[...]
        cols = jax.lax.broadcasted_iota(jnp.int32, hw.shape, 1)
        m1 = jnp.max(hw, axis=1, keepdims=True)
        a1 = jnp.min(jnp.where(hw == m1, cols, N_EXP), axis=1, keepdims=True)
        sel1 = cols == a1
        masked = jnp.where(sel1, -jnp.inf, hw)
        m2 = jnp.max(masked, axis=1, keepdims=True)
        a2 = jnp.min(jnp.where(masked == m2, cols, N_EXP), axis=1,
                     keepdims=True)
        s = m1 + m2 + 1e-8
        g_ref[...] = jnp.where(sel1, m1 / s, 0.0) + jnp.where(
            cols == a2, m2 / s, 0.0)
        out_ref[...] = jnp.zeros_like(out_ref)

    x = x_ref[...].astype(jnp.bfloat16)
    cols = jax.lax.broadcasted_iota(jnp.int32, (N_TOK, N_EXP), 1)
    acc = jnp.zeros((N_TOK, D_MODEL), jnp.float32)
    for j in range(E_BLK):
        e = i * E_BLK + j
        h = jax.lax.dot_general(x, win_ref[j].astype(jnp.bfloat16),
                                (((1,), (1,)), ((), ())),
                                preferred_element_type=jnp.float32)
        h = h + bin_ref[pl.ds(e, 1), :]
        h = h * jax.lax.logistic(h)
        o = jax.lax.dot_general(h.astype(jnp.bfloat16),
                                wout_ref[j].astype(jnp.bfloat16),
                                (((1,), (1,)), ((), ())),
                                preferred_element_type=jnp.float32)
        o = o + bout_ref[pl.ds(e, 1), :]
        g_col = jnp.sum(jnp.where(cols == e, g_ref[...], 0.0), axis=1,
                        keepdims=True)
        acc = acc + g_col * o
    out_ref[...] += acc


@functools.partial(jax.jit, static_argnames=("interpret",))
def kernel(x, hex_weights, W_in, W_out, bias_in, bias_out, interpret=False):
    Bb, Tt, D = x.shape
    x_flat = x.reshape(Bb * Tt, D)
    hex_flat = hex_weights.reshape(Bb * Tt, N_EXP)

    out = pl.pallas_call(
        _moe_kernel,
        grid=(N_EXP // E_BLK,),
        in_specs=[
            pl.BlockSpec((N_TOK, N_EXP), lambda i: (0, 0)),
            pl.BlockSpec((N_TOK, D_MODEL), lambda i: (0, 0)),
            pl.BlockSpec((E_BLK, D_FF, D_MODEL), lambda i: (i, 0, 0)),
            pl.BlockSpec((E_BLK, D_MODEL, D_FF), lambda i: (i, 0, 0)),
            pl.BlockSpec((N_EXP, D_FF), lambda i: (0, 0)),
            pl.BlockSpec((N_EXP, D_MODEL), lambda i: (0, 0)),
        ],
        out_specs=pl.BlockSpec((N_TOK, D_MODEL), lambda i: (0, 0)),
        out_shape=jax.ShapeDtypeStruct((N_TOK, D_MODEL), jnp.float32),
        scratch_shapes=[pltpu.VMEM((N_TOK, N_EXP), jnp.float32)],
        compiler_params=pltpu.CompilerParams(
            dimension_semantics=("arbitrary",)),
        interpret=interpret,
    )(hex_flat, x_flat, W_in, W_out, bias_in, bias_out)
    return out.reshape(Bb, Tt, D)


# speedup vs baseline: 1.6610x; 1.0162x over previous
import functools

import jax
import jax.numpy as jnp
from jax.experimental import pallas as pl
from jax.experimental.pallas import tpu as pltpu

D_MODEL = 768
D_FF = 768
N_EXP = 64
N_TOK = 128
E_BLK = 4


def _moe_kernel(hex_ref, x_ref, win_ref, wout_ref, bin_ref, bout_ref,
                out_ref, g_ref):
    i = pl.program_id(0)

    @pl.when(i == 0)
    def _routing():
        hw = hex_ref[...]
        cols = jax.lax.broadcasted_iota(jnp.int32, hw.shape, 1)
        m1 = jnp.max(hw, axis=1, keepdims=True)
        a1 = jnp.min(jnp.where(hw == m1, cols, N_EXP), axis=1, keepdims=True)
        sel1 = cols == a1
        masked = jnp.where(sel1, -jnp.inf, hw)
        m2 = jnp.max(masked, axis=1, keepdims=True)
        a2 = jnp.min(jnp.where(masked == m2, cols, N_EXP), axis=1,
                     keepdims=True)
        s = m1 + m2 + 1e-8
        g_ref[...] = jnp.where(sel1, m1 / s, 0.0) + jnp.where(
            cols == a2, m2 / s, 0.0)
        out_ref[...] = jnp.zeros_like(out_ref)

    x = x_ref[...].astype(jnp.bfloat16)
    cols = jax.lax.broadcasted_iota(jnp.int32, (N_TOK, N_EXP), 1)
    acc = jnp.zeros((N_TOK, D_MODEL), jnp.float32)
    for j in range(E_BLK):
        e = i * E_BLK + j
        h = jax.lax.dot_general(x, win_ref[j].astype(jnp.bfloat16),
                                (((1,), (1,)), ((), ())),
                                preferred_element_type=jnp.float32)
        h = h + bin_ref[pl.ds(e, 1), :]
        h = h * jax.lax.logistic(h)
        o = jax.lax.dot_general(h.astype(jnp.bfloat16),
                                wout_ref[j].astype(jnp.bfloat16),
                                (((1,), (1,)), ((), ())),
                                preferred_element_type=jnp.float32)
        o = o + bout_ref[pl.ds(e, 1), :]
        g_col = jnp.sum(jnp.where(cols == e, g_ref[...], 0.0), axis=1,
                        keepdims=True)
        acc = acc + g_col * o
    out_ref[...] += acc


@functools.partial(jax.jit, static_argnames=("interpret",))
def kernel(x, hex_weights, W_in, W_out, bias_in, bias_out, interpret=False):
    Bb, Tt, D = x.shape
    x_flat = x.reshape(Bb * Tt, D)
    hex_flat = hex_weights.reshape(Bb * Tt, N_EXP)

    out = pl.pallas_call(
        _moe_kernel,
        grid=(N_EXP // E_BLK,),
        in_specs=[
            pl.BlockSpec((N_TOK, N_EXP), lambda i: (0, 0)),
            pl.BlockSpec((N_TOK, D_MODEL), lambda i: (0, 0)),
            pl.BlockSpec((E_BLK, D_FF, D_MODEL), lambda i: (i, 0, 0)),
            pl.BlockSpec((E_BLK, D_MODEL, D_FF), lambda i: (i, 0, 0)),
            pl.BlockSpec((N_EXP, D_FF), lambda i: (0, 0)),
            pl.BlockSpec((N_EXP, D_MODEL), lambda i: (0, 0)),
        ],
        out_specs=pl.BlockSpec((N_TOK, D_MODEL), lambda i: (0, 0)),
        out_shape=jax.ShapeDtypeStruct((N_TOK, D_MODEL), jnp.float32),
        scratch_shapes=[pltpu.VMEM((N_TOK, N_EXP), jnp.float32)],
        compiler_params=pltpu.CompilerParams(
            dimension_semantics=("arbitrary",)),
        interpret=interpret,
    )(hex_flat, x_flat, W_in, W_out, bias_in, bias_out)
    return out.reshape(Bb, Tt, D)
